# baseline (speedup 1.0000x reference)
import functools

import jax
import jax.numpy as jnp
import numpy as np
from jax.experimental import pallas as pl
from jax.experimental.pallas import tpu as pltpu

BN = 80
LOG2E = float(np.log2(np.e))


def _gat_kernel(n, x_ref, a_ref, d_ref, w_ref, as_ref, an_ref, b_ref,
                out_ref, abuf_ref, s_ref, t_ref, tc_ref, sem_ref):
    n_k = n // BN

    def _copy(k, slot):
        return pltpu.make_async_copy(
            a_ref.at[pl.ds(k * BN, BN), :],
            abuf_ref.at[slot],
            sem_ref.at[slot])

    wst = jnp.concatenate([as_ref[...], an_ref[...]], axis=1)
    wst = jnp.dot(w_ref[...], wst,
                  preferred_element_type=jnp.float32) * LOG2E
    st = jnp.dot(x_ref[...], wst, preferred_element_type=jnp.float32)
    s_ref[...] = st[:, 0:1]
    tc_ref[...] = st[:, 1:2]
    t_ref[...] = st[:, 1:2].reshape(1, n)

    _copy(0, 0).start()

    def _body(k, _):
        slot = jax.lax.rem(k, 2)

        @pl.when(k + 1 < n_k)
        def _prefetch():
            _copy(k + 1, 1 - slot).start()

        _copy(k, slot).wait()

        s_blk = s_ref[pl.ds(k * BN, BN), :]
        z = s_blk + t_ref[...]
        logit = jnp.maximum(z, 0.2 * z)
        p = jnp.exp2(logit) * abuf_ref[slot]
        l = jnp.sum(p, axis=1, keepdims=True)
        acc = jnp.dot(p, x_ref[...],
                      preferred_element_type=jnp.float32)

        d_blk = d_ref[pl.ds(k * BN, BN), :]
        t_self = tc_ref[pl.ds(k * BN, BN), :]
        zs = s_blk + t_self
        w_self = (1.0 - d_blk) * jnp.exp2(jnp.maximum(zs, 0.2 * zs))
        l = l + w_self
        x_row = x_ref[pl.ds(k * BN, BN), :]
        acc = acc + w_self * x_row
        out_ref[pl.ds(k * BN, BN), :] = jnp.dot(
            acc / l, w_ref[...],
            preferred_element_type=jnp.float32) + b_ref[...]
        return 0

    jax.lax.fori_loop(0, n_k, _body, 0)


@jax.jit
def kernel(x, a, kernel, attn_kernel_self, attn_kernel_neighs, bias):
    n, i_dim = x.shape
    o_dim = kernel.shape[2]
    w = kernel.reshape(i_dim, o_dim)
    a_s = attn_kernel_self.reshape(o_dim, 1)
    a_n = attn_kernel_neighs.reshape(o_dim, 1)
    b = bias.reshape(1, o_dim)
    d = jnp.diagonal(a).reshape(n, 1)

    out = pl.pallas_call(
        functools.partial(_gat_kernel, n),
        in_specs=[
            pl.BlockSpec(memory_space=pltpu.MemorySpace.VMEM),
            pl.BlockSpec(memory_space=pl.ANY),
            pl.BlockSpec(memory_space=pltpu.MemorySpace.VMEM),
            pl.BlockSpec(memory_space=pltpu.MemorySpace.VMEM),
            pl.BlockSpec(memory_space=pltpu.MemorySpace.VMEM),
            pl.BlockSpec(memory_space=pltpu.MemorySpace.VMEM),
            pl.BlockSpec(memory_space=pltpu.MemorySpace.VMEM),
        ],
        out_specs=pl.BlockSpec(memory_space=pltpu.MemorySpace.VMEM),
        out_shape=jax.ShapeDtypeStruct((n, o_dim), jnp.float32),
        scratch_shapes=[
            pltpu.VMEM((2, BN, n), jnp.float32),
            pltpu.VMEM((n, 1), jnp.float32),
            pltpu.VMEM((1, n), jnp.float32),
            pltpu.VMEM((n, 1), jnp.float32),
            pltpu.SemaphoreType.DMA((2,)),
        ],
    )(x, a, d, w, a_s, a_n, b)
    return out


# speedup vs baseline: 1.1535x; 1.1535x over previous
import functools

import jax
import jax.numpy as jnp
import numpy as np
from jax.experimental import pallas as pl
from jax.experimental.pallas import tpu as pltpu

BN = 1024
BM = 1024
LOG2E = float(np.log2(np.e))


def _flash_kernel(n_real, n_col_blocks,
                  x_ref, xa_ref, a_ref, d_ref, w_ref, as_ref, an_ref, b_ref,
                  out_ref, acc_ref, s_ref, t_ref, tc_ref):
    i = pl.program_id(0)
    j = pl.program_id(1)

    @pl.when(jnp.logical_and(i == 0, j == 0))
    def _init_globals():
        wst = jnp.concatenate([as_ref[...], an_ref[...]], axis=1)
        wst = jnp.dot(w_ref[...], wst,
                      preferred_element_type=jnp.float32) * LOG2E
        st = jnp.dot(x_ref[...], wst, preferred_element_type=jnp.float32)
        s_ref[...] = st[:, 0:1]
        tc_ref[...] = st[:, 1:2]
        t_ref[...] = st[:, 1:2].reshape(1, -1)
        if n_real % BM:
            t_ref[:, n_real:] = jnp.full((1, t_ref.shape[1] - n_real), -1e30,
                                         jnp.float32)

    @pl.when(j == 0)
    def _init_row_block():
        acc_ref[...] = jnp.zeros_like(acc_ref)

    s_blk = s_ref[pl.ds(i * BN, BN), :]
    t_blk = t_ref[:, pl.ds(j * BM, BM)]
    z = s_blk + t_blk
    logit = jnp.maximum(z, 0.2 * z)
    e = jnp.exp2(logit).astype(jnp.bfloat16)
    p = e * a_ref[...].astype(jnp.bfloat16)
    xa_col = xa_ref[pl.ds(j * BM, BM), :]
    acc_ref[...] += jnp.dot(p, xa_col,
                            preferred_element_type=jnp.float32)

    @pl.when(j == n_col_blocks - 1)
    def _finalize():
        d_blk = d_ref[pl.ds(i * BN, BN), :]
        t_self = tc_ref[pl.ds(i * BN, BN), :]
        zs = s_blk + t_self
        w_self = (1.0 - d_blk) * jnp.exp2(jnp.maximum(zs, 0.2 * zs))
        xa_row = xa_ref[pl.ds(i * BN, BN), :].astype(jnp.float32)
        acc = acc_ref[...] + w_self * xa_row
        o_dim = w_ref.shape[0]
        l = acc[:, o_dim:o_dim + 1]
        out_ref[...] = jnp.dot(acc[:, :o_dim] / l, w_ref[...],
                               preferred_element_type=jnp.float32) + b_ref[...]


@jax.jit
def kernel(x, a, kernel, attn_kernel_self, attn_kernel_neighs, bias):
    n, i_dim = x.shape
    o_dim = kernel.shape[2]
    w = kernel.reshape(i_dim, o_dim)
    a_s = attn_kernel_self.reshape(o_dim, 1)
    a_n = attn_kernel_neighs.reshape(o_dim, 1)
    b = bias.reshape(1, o_dim)

    n_row_blocks = pl.cdiv(n, BN)
    n_col_blocks = pl.cdiv(n, BM)
    n_pad = max(n_row_blocks * BN, n_col_blocks * BM)
    x_p = jnp.pad(x, ((0, n_pad - n), (0, 0)))
    x_aug = jnp.concatenate(
        [x_p, jnp.ones((n_pad, 1), jnp.float32),
         jnp.zeros((n_pad, i_dim - 1), jnp.float32)],
        axis=1).astype(jnp.bfloat16)
    d_p = jnp.pad(jnp.diagonal(a), (0, n_pad - n),
                  constant_values=1.0).reshape(n_pad, 1)

    grid = (n_row_blocks, n_col_blocks)
    out = pl.pallas_call(
        functools.partial(_flash_kernel, n, n_col_blocks),
        grid=grid,
        in_specs=[
            pl.BlockSpec((n_pad, i_dim), lambda i, j: (0, 0)),
            pl.BlockSpec((n_pad, 2 * i_dim), lambda i, j: (0, 0)),
            pl.BlockSpec((BN, BM), lambda i, j: (i, j)),
            pl.BlockSpec((BN, 1), lambda i, j: (i, 0)),
            pl.BlockSpec((i_dim, o_dim), lambda i, j: (0, 0)),
            pl.BlockSpec((o_dim, 1), lambda i, j: (0, 0)),
            pl.BlockSpec((o_dim, 1), lambda i, j: (0, 0)),
            pl.BlockSpec((1, o_dim), lambda i, j: (0, 0)),
        ],
        out_specs=pl.BlockSpec((BN, o_dim), lambda i, j: (i, 0)),
        out_shape=jax.ShapeDtypeStruct((n, o_dim), jnp.float32),
        scratch_shapes=[
            pltpu.VMEM((BN, 2 * i_dim), jnp.float32),
            pltpu.VMEM((n_pad, 1), jnp.float32),
            pltpu.VMEM((1, n_pad), jnp.float32),
            pltpu.VMEM((n_pad, 1), jnp.float32),
        ],
        compiler_params=pltpu.CompilerParams(
            dimension_semantics=("arbitrary", "arbitrary")),
    )(x_p, x_aug, a, d_p, w, a_s, a_n, b)
    return out
